# Initial kernel scaffold; baseline (speedup 1.0000x reference)
import jax
import jax.numpy as jnp
from jax.experimental import pallas as pl


def kernel(input, W):
    raise NotImplementedError("write your pallas kernel here")


# speedup vs baseline: 5.7236x; 5.7236x over previous
import functools

import jax
import jax.numpy as jnp
from jax import lax
from jax.experimental import pallas as pl
from jax.experimental.pallas import tpu as pltpu
from jax.experimental.pallas import tpu_sc as plsc

B, T = 16384, 200
V, D = 32, 8
M = B * T
NC, NS, L = 2, 16, 16
NW = NC * NS
PER_W = M // NW
CHUNK = 4096
N_CHUNKS = PER_W // CHUNK


@functools.partial(
    pl.kernel,
    mesh=plsc.VectorSubcoreMesh(core_axis_name="c", subcore_axis_name="s"),
    out_type=jax.ShapeDtypeStruct((M * D,), jnp.float32),
    compiler_params=pltpu.CompilerParams(needs_layout_passes=False),
    scratch_types=[
        pltpu.VMEM((V * D,), jnp.float32),
        pltpu.VMEM((CHUNK,), jnp.int32),
        pltpu.VMEM((CHUNK * D,), jnp.float32),
    ],
)
def _gather_kernel(idx_hbm, w_hbm, out_hbm, tab_v, idx_v, out_v):
    wid = lax.axis_index("s") * NC + lax.axis_index("c")
    base = wid * PER_W
    pltpu.sync_copy(w_hbm, tab_v)
    iota8 = lax.broadcasted_iota(jnp.int32, (L,), 0) * D

    def chunk_body(ci, carry):
        cbase = base + ci * CHUNK
        pltpu.sync_copy(idx_hbm.at[pl.ds(cbase, CHUNK)], idx_v)

        def grp(gi, carry2):
            b = gi * L
            iv = idx_v[pl.ds(b, L)]
            g = iv * D
            p = iota8 + b * D
            for j in range(D):
                val = plsc.load_gather(tab_v, [g + j])
                plsc.store_scatter(out_v, [p + j], val)
            return carry2

        lax.fori_loop(0, CHUNK // L, grp, 0)
        pltpu.sync_copy(out_v, out_hbm.at[pl.ds(cbase * D, CHUNK * D)])
        return carry

    lax.fori_loop(0, N_CHUNKS, chunk_body, 0)


def kernel(input, W):
    out_flat = _gather_kernel(input.reshape(-1), W.reshape(-1))
    return out_flat.reshape(B, T, D)


# speedup vs baseline: 6.4010x; 1.1183x over previous
import functools

import jax
import jax.numpy as jnp
from jax import lax
from jax.experimental import pallas as pl
from jax.experimental.pallas import tpu as pltpu
from jax.experimental.pallas import tpu_sc as plsc

B, T = 16384, 200
V, D = 32, 8
M = B * T
NC, NS, L = 2, 16, 16
NW = NC * NS
PER_W = M // NW
CHUNK = 4096
N_CHUNKS = PER_W // CHUNK


@functools.partial(
    pl.kernel,
    mesh=plsc.VectorSubcoreMesh(core_axis_name="c", subcore_axis_name="s"),
    out_type=jax.ShapeDtypeStruct((M * D,), jnp.float32),
    compiler_params=pltpu.CompilerParams(needs_layout_passes=False),
    scratch_types=[
        pltpu.VMEM((V * D,), jnp.float32),
        pltpu.VMEM((CHUNK,), jnp.int32),
        pltpu.VMEM((CHUNK * D,), jnp.float32),
    ],
)
def _gather_kernel(idx_hbm, w_hbm, out_hbm, tab_v, idx_v, out_v):
    wid = lax.axis_index("s") * NC + lax.axis_index("c")
    base = wid * PER_W
    pltpu.sync_copy(w_hbm, tab_v)
    iota8 = lax.broadcasted_iota(jnp.int32, (L,), 0) * D

    def chunk_body(ci, carry):
        cbase = base + ci * CHUNK
        pltpu.sync_copy(idx_hbm.at[pl.ds(cbase, CHUNK)], idx_v)

        @plsc.parallel_loop(0, CHUNK // L, 1, unroll=8)
        def grp(gi):
            b = gi * L
            iv = idx_v[pl.ds(b, L)]
            g = iv * D
            p = iota8 + b * D
            for j in range(D):
                val = plsc.load_gather(tab_v, [g + j])
                plsc.store_scatter(out_v, [p + j], val)
        pltpu.sync_copy(out_v, out_hbm.at[pl.ds(cbase * D, CHUNK * D)])
        return carry

    lax.fori_loop(0, N_CHUNKS, chunk_body, 0)


def kernel(input, W):
    out_flat = _gather_kernel(input.reshape(-1), W.reshape(-1))
    return out_flat.reshape(B, T, D)


# speedup vs baseline: 92.3810x; 14.4323x over previous
import functools

import jax
import jax.numpy as jnp
from jax import lax
from jax.experimental import pallas as pl
from jax.experimental.pallas import tpu as pltpu
from jax.experimental.pallas import tpu_sc as plsc

B, T = 16384, 200
V, D = 32, 8
M = B * T
NC, NS, L = 2, 16, 16
NW = NC * NS
GSZ = 1024
NG = M // GSZ
GPW = NG // NW
VPG = GSZ // L


@functools.partial(
    pl.kernel,
    mesh=plsc.VectorSubcoreMesh(core_axis_name="c", subcore_axis_name="s"),
    out_type=jax.ShapeDtypeStruct((M * D,), jnp.float32),
    compiler_params=pltpu.CompilerParams(needs_layout_passes=False),
    scratch_types=[
        pltpu.VMEM((V * D,), jnp.float32),
        pltpu.VMEM((GSZ,), jnp.int32),
        pltpu.VMEM((GSZ,), jnp.int32),
        pltpu.VMEM((GSZ * D,), jnp.float32),
        pltpu.VMEM((GSZ * D,), jnp.float32),
        pltpu.SemaphoreType.DMA,
        pltpu.SemaphoreType.DMA,
        pltpu.SemaphoreType.DMA,
        pltpu.SemaphoreType.DMA,
    ],
)
def _gather_kernel(idx_hbm, w_hbm, out_hbm, tab_v, idx_v0, idx_v1,
                   out_v0, out_v1, isem0, isem1, osem0, osem1):
    wid = lax.axis_index("s") * NC + lax.axis_index("c")
    g0 = wid * GPW
    pltpu.sync_copy(w_hbm, tab_v)

    idx_bufs = (idx_v0, idx_v1)
    out_bufs = (out_v0, out_v1)
    isems = (isem0, isem1)
    osems = (osem0, osem1)

    def in_copy(gi, sel):
        t = gi // 16
        btg = gi % 16
        src = idx_hbm.at[pl.ds(t * B + btg * GSZ, GSZ)]
        return pltpu.make_async_copy(src, idx_bufs[sel], isems[sel])

    def out_copy(gi, sel):
        t = gi // 16
        btg = gi % 16
        dst = out_hbm.at[pl.ds(t * (B * D) + btg * (GSZ * D), GSZ * D)]
        return pltpu.make_async_copy(out_bufs[sel], dst, osems[sel])

    def compute(sel):
        idx_v = idx_bufs[sel]
        out_v = out_bufs[sel]

        @plsc.parallel_loop(0, VPG, 1, unroll=4)
        def vr(k):
            iv = idx_v[pl.ds(k * L, L)]
            g = iv * D
            base = ((k >> 3) << 10) + ((k & 7) << 4)
            for j in range(D):
                val = plsc.load_gather(tab_v, [g + j])
                out_v[pl.ds(base + j * 128, L)] = val

    in_copy(g0, 0).start()

    def pair_body(p, carry):
        i0 = g0 + 2 * p
        i1 = i0 + 1
        in_copy(i1, 1).start()
        in_copy(i0, 0).wait()

        @pl.when(p > 0)
        def _():
            out_copy(i0 - 2, 0).wait()

        compute(0)
        out_copy(i0, 0).start()

        @pl.when(p < GPW // 2 - 1)
        def _():
            in_copy(i0 + 2, 0).start()

        in_copy(i1, 1).wait()

        @pl.when(p > 0)
        def _():
            out_copy(i1 - 2, 1).wait()

        compute(1)
        out_copy(i1, 1).start()
        return carry

    lax.fori_loop(0, GPW // 2, pair_body, 0)
    out_copy(g0 + GPW - 2, 0).wait()
    out_copy(g0 + GPW - 1, 1).wait()


def kernel(input, W):
    idx_t = input.T.reshape(-1)
    out_flat = _gather_kernel(idx_t, W.reshape(-1))
    return (
        out_flat.reshape(T, B // 128, D, 128)
        .transpose(1, 3, 0, 2)
        .reshape(B, T, D)
    )


# speedup vs baseline: 97.2109x; 1.0523x over previous
import functools

import jax
import jax.numpy as jnp
from jax import lax
from jax.experimental import pallas as pl
from jax.experimental.pallas import tpu as pltpu
from jax.experimental.pallas import tpu_sc as plsc

B, T = 16384, 200
V, D = 32, 8
M = B * T
NC, NS, L = 2, 16, 16
NW = NC * NS
GSZ = 2048
NG = M // GSZ
GPW = NG // NW
GPT = B // GSZ
VPG = GSZ // L


@functools.partial(
    pl.kernel,
    mesh=plsc.VectorSubcoreMesh(core_axis_name="c", subcore_axis_name="s"),
    out_type=jax.ShapeDtypeStruct((M * D,), jnp.float32),
    compiler_params=pltpu.CompilerParams(needs_layout_passes=False),
    scratch_types=[
        pltpu.VMEM((V * D,), jnp.float32),
        pltpu.VMEM((GSZ,), jnp.int32),
        pltpu.VMEM((GSZ,), jnp.int32),
        pltpu.VMEM((GSZ * D,), jnp.float32),
        pltpu.VMEM((GSZ * D,), jnp.float32),
        pltpu.SemaphoreType.DMA,
        pltpu.SemaphoreType.DMA,
        pltpu.SemaphoreType.DMA,
        pltpu.SemaphoreType.DMA,
    ],
)
def _gather_kernel(idx_hbm, w_hbm, out_hbm, tab_v, idx_v0, idx_v1,
                   out_v0, out_v1, isem0, isem1, osem0, osem1):
    wid = lax.axis_index("s") * NC + lax.axis_index("c")
    g0 = wid * GPW
    pltpu.sync_copy(w_hbm, tab_v)

    idx_bufs = (idx_v0, idx_v1)
    out_bufs = (out_v0, out_v1)
    isems = (isem0, isem1)
    osems = (osem0, osem1)

    def in_copy(gi, sel):
        t = gi // GPT
        btg = gi % GPT
        src = idx_hbm.at[pl.ds(t * B + btg * GSZ, GSZ)]
        return pltpu.make_async_copy(src, idx_bufs[sel], isems[sel])

    def out_copy(gi, sel):
        t = gi // GPT
        btg = gi % GPT
        dst = out_hbm.at[pl.ds(t * (B * D) + btg * (GSZ * D), GSZ * D)]
        return pltpu.make_async_copy(out_bufs[sel], dst, osems[sel])

    def compute(sel):
        idx_v = idx_bufs[sel]
        out_v = out_bufs[sel]

        @plsc.parallel_loop(0, VPG, 1, unroll=4)
        def vr(k):
            iv = idx_v[pl.ds(k * L, L)]
            g = iv * D
            base = ((k >> 3) << 10) + ((k & 7) << 4)
            for j in range(D):
                val = plsc.load_gather(tab_v, [g + j])
                out_v[pl.ds(base + j * 128, L)] = val

    in_copy(g0, 0).start()

    def pair_body(p, carry):
        i0 = g0 + 2 * p
        i1 = i0 + 1
        in_copy(i1, 1).start()
        in_copy(i0, 0).wait()

        @pl.when(p > 0)
        def _():
            out_copy(i0 - 2, 0).wait()

        compute(0)
        out_copy(i0, 0).start()

        @pl.when(p < GPW // 2 - 1)
        def _():
            in_copy(i0 + 2, 0).start()

        in_copy(i1, 1).wait()

        @pl.when(p > 0)
        def _():
            out_copy(i1 - 2, 1).wait()

        compute(1)
        out_copy(i1, 1).start()
        return carry

    lax.fori_loop(0, GPW // 2, pair_body, 0)
    out_copy(g0 + GPW - 2, 0).wait()
    out_copy(g0 + GPW - 1, 1).wait()


def kernel(input, W):
    idx_t = input.T.reshape(-1)
    out_flat = _gather_kernel(idx_t, W.reshape(-1))
    return (
        out_flat.reshape(T, B // 128, D, 128)
        .transpose(1, 3, 0, 2)
        .reshape(B, T, D)
    )
